# baseline (speedup 1.0000x reference)
import functools
import jax
import jax.numpy as jnp
from jax import lax
from jax.experimental import pallas as pl
from jax.experimental.pallas import tpu as pltpu

_B = 4 * 2048
_D = 1280
_V = 64
_R = 512


def _table_body(emb_ref, w_ref, b_ref, out_ref):
    w_sum = w_ref[:, :512] + w_ref[:, 512:]
    acc = jax.lax.dot_general(
        emb_ref[:], w_sum,
        dimension_numbers=(((1,), (1,)), ((), ())),
        preferred_element_type=jnp.float32,
    )
    out_ref[:] = acc + b_ref[:]


def _compute_table(emb, w, b):
    emb_pad = jnp.zeros((_V, 512), jnp.float32).at[:33].set(emb)
    return pl.pallas_call(
        _table_body,
        out_shape=jax.ShapeDtypeStruct((_V, _D), jnp.float32),
    )(emb_pad, w, b.reshape(1, _D))


def _expand_body(tok_ref, table_ref, out_ref):
    tok = tok_ref[0]
    oh = (tok[0, :, None] == lax.broadcasted_iota(jnp.int32, (1, _V), 1)
          ).astype(jnp.float32)
    out_ref[:] = jax.lax.dot_general(
        oh, table_ref[:],
        dimension_numbers=(((1,), (0,)), ((), ())),
        preferred_element_type=jnp.float32)


_expand = pl.pallas_call(
    _expand_body,
    grid=(_B // _R,),
    in_specs=[
        pl.BlockSpec((1, 1, _R), lambda i: (i, 0, 0)),
        pl.BlockSpec((_V, _D), lambda i: (0, 0)),
    ],
    out_specs=pl.BlockSpec((_R, _D), lambda i: (i, 0)),
    out_shape=jax.ShapeDtypeStruct((_B, _D), jnp.float32),
)


def kernel(tokens, emb, W, b):
    table = _compute_table(emb, W, b)
    tok = tokens.astype(jnp.int32).reshape(_B // _R, 1, _R)
    out = _expand(tok, table)
    return out.reshape(tokens.shape[0], tokens.shape[1], _D)
